# Initial kernel scaffold; baseline (speedup 1.0000x reference)
import jax
import jax.numpy as jnp
from jax.experimental import pallas as pl


def kernel(user_cat, user_num, ctx_cat, hist_ids, hist_mask, item_cat, item_num, user_tables, ctx_tables, item_table0, item_tables_rest, Wun, bun, Win, bin, Wu1, bu1, Wu2, bu2, Wi1, bi1, Wi2, bi2):
    raise NotImplementedError("write your pallas kernel here")


# speedup vs baseline: 1.0052x; 1.0052x over previous
import functools

import jax
import jax.numpy as jnp
from jax import lax
from jax.experimental import pallas as pl
from jax.experimental.pallas import tpu as pltpu
from jax.experimental.pallas import tpu_sc as plsc

_B = 4096
_D = 16
_NU = 23
_NC = 3
_NI = 8
_HL = 200
_VU = 100000
_VC = 1000
_VI0 = 1000000
_VIR = 100000
_UNUM = 4
_INUM = 6
_HID = 128
_TOW = 64
_UIN = _NU * _D + _NC * _D + 2 * _D
_IIN = _NI * _D + _D

_NW = 32
_RPT = _B // _NW
_HLP = 208


def _sc_body(ucat, ccat, icat, histf, hmaskf, unum, inum, wun, bun2, win,
             bin2, utf, ctf, it0, irf, x_out, y_out,
             idx_v, ids_v, hmask_v, hrows_v, emb_v, pool_v, num_v,
             un_v, in_v, wun_v, bun_v, win_v, bin_v, sem):
    info = plsc.get_sparse_core_info()
    wid = lax.axis_index("s") * info.num_cores + lax.axis_index("c")
    b0 = wid * _RPT
    bs = pl.ds(b0, _RPT)

    pltpu.sync_copy(wun, wun_v)
    pltpu.sync_copy(bun2, bun_v)
    pltpu.sync_copy(win, win_v)
    pltpu.sync_copy(bin2, bin_v)
    pltpu.sync_copy(unum.at[pl.ds(b0 * _D, _RPT * _D)], un_v)
    pltpu.sync_copy(inum.at[pl.ds(b0 * _D, _RPT * _D)], in_v)

    def unum_body(r, carry):
        uvec = un_v[pl.ds(pl.multiple_of(r * _D, _D), _D)]
        acc = bun_v[...]
        for k in range(_UNUM):
            acc = acc + uvec[k] * wun_v[pl.ds(k * _D, _D)]
        num_v[r, :] = acc
        return carry
    lax.fori_loop(0, _RPT, unum_body, 0)
    pltpu.sync_copy(num_v, x_out.at[bs, pl.ds(26 * _D, _D)])

    def inum_body(r, carry):
        ivec = in_v[pl.ds(pl.multiple_of(r * _D, _D), _D)]
        acc = bin_v[...]
        for k in range(_INUM):
            acc = acc + ivec[k] * win_v[pl.ds(k * _D, _D)]
        num_v[r, :] = acc
        return carry
    lax.fori_loop(0, _RPT, inum_body, 0)
    pltpu.sync_copy(num_v, y_out.at[bs, pl.ds(_NI * _D, _D)])

    def gather_feature(src_slice, offset, table, dst):
        pltpu.sync_copy(src_slice, idx_v)
        if offset:
            for c in range(_RPT // 16):
                sl = pl.ds(c * 16, 16)
                idx_v[sl] = idx_v[sl] + offset
        pltpu.async_copy(table.at[idx_v], emb_v, sem).wait()
        pltpu.sync_copy(emb_v, dst)

    for f in range(_NU):
        gather_feature(ucat.at[pl.ds(f * _B + b0, _RPT)], f * _VU, utf,
                       x_out.at[bs, pl.ds(f * _D, _D)])
    for f in range(_NC):
        gather_feature(ccat.at[pl.ds(f * _B + b0, _RPT)], f * _VC, ctf,
                       x_out.at[bs, pl.ds((_NU + f) * _D, _D)])
    gather_feature(icat.at[pl.ds(b0, _RPT)], 0, it0,
                   y_out.at[bs, pl.ds(0, _D)])
    for f in range(1, _NI):
        gather_feature(icat.at[pl.ds(f * _B + b0, _RPT)], (f - 1) * _VIR,
                       irf, y_out.at[bs, pl.ds(f * _D, _D)])

    zv = jnp.zeros((16,), jnp.float32)
    hmask_v[pl.ds(192, 16)] = zv
    for t in range(_HL, _HLP):
        hrows_v[t, :] = zv

    def hist_body(r, carry):
        b = b0 + r
        h0 = pl.multiple_of(b * _HL, 8)
        pltpu.sync_copy(histf.at[pl.ds(h0, 128)], ids_v.at[pl.ds(0, 128)])
        pltpu.sync_copy(histf.at[pl.ds(h0 + 128, _HL - 128)],
                        ids_v.at[pl.ds(128, _HL - 128)])
        pltpu.sync_copy(hmaskf.at[pl.ds(h0, _HL)], hmask_v.at[pl.ds(0, _HL)])
        cp1 = pltpu.async_copy(it0.at[ids_v.at[pl.ds(0, 128)]],
                               hrows_v.at[pl.ds(0, 128)], sem)
        cp2 = pltpu.async_copy(it0.at[ids_v.at[pl.ds(128, _HL - 128)]],
                               hrows_v.at[pl.ds(128, _HL - 128)], sem)
        cp1.wait()
        cp2.wait()

        def acc_body(c, carry2):
            a, ms = carry2
            mvec = hmask_v[pl.ds(pl.multiple_of(c * 16, 16), 16)]
            base = c * 16
            for j in range(16):
                mj = mvec[j]
                a = a + hrows_v[base + j, :] * mj
                ms = ms + mj
            return (a, ms)
        a, ms = lax.fori_loop(0, _HLP // 16, acc_body,
                              (zv, jnp.float32(0.0)))
        pool_v[r, :] = a / jnp.maximum(ms, 1e-6)
        return carry
    lax.fori_loop(0, _RPT, hist_body, 0)
    pltpu.sync_copy(pool_v, x_out.at[bs, pl.ds(27 * _D, _D)])


def _tc_body(x_ref, y_ref, wu1, bu1, wu2, bu2, wi1, bi1, wi2, bi2,
             u_ref, i_ref):
    f32 = jnp.float32
    xb = x_ref[...]
    h = jnp.maximum(
        jnp.dot(xb, wu1[...], preferred_element_type=f32) + bu1[...], 0.0)
    uu = jnp.dot(h, wu2[...], preferred_element_type=f32) + bu2[...]
    n = jnp.sqrt(jnp.sum(uu * uu, axis=-1, keepdims=True))
    u_ref[...] = uu / jnp.maximum(n, 1e-12)

    yb = y_ref[...]
    h2 = jnp.maximum(
        jnp.dot(yb, wi1[...], preferred_element_type=f32) + bi1[...], 0.0)
    ii = jnp.dot(h2, wi2[...], preferred_element_type=f32) + bi2[...]
    n2 = jnp.sqrt(jnp.sum(ii * ii, axis=-1, keepdims=True))
    i_ref[...] = ii / jnp.maximum(n2, 1e-12)


def kernel(user_cat, user_num, ctx_cat, hist_ids, hist_mask, item_cat,
           item_num, user_tables, ctx_tables, item_table0, item_tables_rest,
           Wun, bun, Win, bin, Wu1, bu1, Wu2, bu2, Wi1, bi1, Wi2, bi2):
    f32 = jnp.float32
    ucat_f = user_cat.T.astype(jnp.int32).reshape(-1)
    ccat_f = ctx_cat.T.astype(jnp.int32).reshape(-1)
    icat_f = item_cat.T.astype(jnp.int32).reshape(-1)
    hist_flat = hist_ids.reshape(-1).astype(jnp.int32)
    hmask_flat = hist_mask.reshape(-1)
    unum_pad = jnp.pad(user_num, ((0, 0), (0, _D - _UNUM))).reshape(-1)
    inum_pad = jnp.pad(item_num, ((0, 0), (0, _D - _INUM))).reshape(-1)
    ut_flat = user_tables.reshape(_NU * _VU, _D)
    ct_flat = ctx_tables.reshape(_NC * _VC, _D)
    ir_flat = item_tables_rest.reshape((_NI - 1) * _VIR, _D)

    mesh = plsc.VectorSubcoreMesh(core_axis_name="c", subcore_axis_name="s")
    sc = functools.partial(
        pl.kernel,
        mesh=mesh,
        compiler_params=pltpu.CompilerParams(use_tc_tiling_on_sc=False),
        out_type=[jax.ShapeDtypeStruct((_B, _UIN), f32),
                  jax.ShapeDtypeStruct((_B, _IIN), f32)],
        scratch_types=[
            pltpu.VMEM((_RPT,), jnp.int32),
            pltpu.VMEM((256,), jnp.int32),
            pltpu.VMEM((_HLP,), f32),
            pltpu.VMEM((_HLP, _D), f32),
            pltpu.VMEM((_RPT, _D), f32),
            pltpu.VMEM((_RPT, _D), f32),
            pltpu.VMEM((_RPT, _D), f32),
            pltpu.VMEM((_RPT * _D,), f32),
            pltpu.VMEM((_RPT * _D,), f32),
            pltpu.VMEM((_UNUM * _D,), f32),
            pltpu.VMEM((_D,), f32),
            pltpu.VMEM((_INUM * _D,), f32),
            pltpu.VMEM((_D,), f32),
            pltpu.SemaphoreType.DMA,
        ],
    )(_sc_body)
    x, y = sc(ucat_f, ccat_f, icat_f, hist_flat, hmask_flat, unum_pad,
              inum_pad, Wun.reshape(-1), bun, Win.reshape(-1), bin,
              ut_flat, ct_flat, item_table0, ir_flat)

    bm = 1024
    grid = _B // bm
    full = lambda i: (0, 0)
    u, i = pl.pallas_call(
        _tc_body,
        grid=(grid,),
        in_specs=[
            pl.BlockSpec((bm, _UIN), lambda i: (i, 0)),
            pl.BlockSpec((bm, _IIN), lambda i: (i, 0)),
            pl.BlockSpec((_UIN, _HID), full),
            pl.BlockSpec((1, _HID), full),
            pl.BlockSpec((_HID, _TOW), full),
            pl.BlockSpec((1, _TOW), full),
            pl.BlockSpec((_IIN, _HID), full),
            pl.BlockSpec((1, _HID), full),
            pl.BlockSpec((_HID, _TOW), full),
            pl.BlockSpec((1, _TOW), full),
        ],
        out_specs=[pl.BlockSpec((bm, _TOW), lambda i: (i, 0)),
                   pl.BlockSpec((bm, _TOW), lambda i: (i, 0))],
        out_shape=[jax.ShapeDtypeStruct((_B, _TOW), f32),
                   jax.ShapeDtypeStruct((_B, _TOW), f32)],
    )(x, y, Wu1, bu1.reshape(1, _HID), Wu2, bu2.reshape(1, _TOW),
      Wi1, bi1.reshape(1, _HID), Wi2, bi2.reshape(1, _TOW))
    return (u, i)


# speedup vs baseline: 1.1901x; 1.1839x over previous
import functools

import jax
import jax.numpy as jnp
from jax import lax
from jax.experimental import pallas as pl
from jax.experimental.pallas import tpu as pltpu
from jax.experimental.pallas import tpu_sc as plsc

_B = 4096
_D = 16
_NU = 23
_NC = 3
_NI = 8
_HL = 200
_VU = 100000
_VC = 1000
_VI0 = 1000000
_VIR = 100000
_UNUM = 4
_INUM = 6
_HID = 128
_TOW = 64
_UIN = _NU * _D + _NC * _D + 2 * _D
_IIN = _NI * _D + _D

_NW = 32
_RPT = _B // _NW
_RPS = 4
_NST = _RPT // _RPS
_SID = _RPS * _HL


def _sc_body(ucat, ccat, icat, histf, hmaskf, unum, inum, wun, bun2, win,
             bin2, utf, ctf, it0, irf, x_out, y_out,
             idxA, idxB, embA, embB, ids_all, hmask_all, hrA, hrB,
             pool_v, num_v, un_v, in_v, wun_v, bun_v, win_v, bin_v,
             semA, semB):
    info = plsc.get_sparse_core_info()
    wid = lax.axis_index("s") * info.num_cores + lax.axis_index("c")
    b0 = wid * _RPT
    bs = pl.ds(b0, _RPT)

    pltpu.sync_copy(wun, wun_v)
    pltpu.sync_copy(bun2, bun_v)
    pltpu.sync_copy(win, win_v)
    pltpu.sync_copy(bin2, bin_v)
    pltpu.sync_copy(unum.at[pl.ds(b0 * _D, _RPT * _D)], un_v)
    pltpu.sync_copy(inum.at[pl.ds(b0 * _D, _RPT * _D)], in_v)

    def unum_body(r, carry):
        uvec = un_v[pl.ds(pl.multiple_of(r * _D, _D), _D)]
        acc = bun_v[...]
        for k in range(_UNUM):
            acc = acc + uvec[k] * wun_v[pl.ds(k * _D, _D)]
        num_v[r, :] = acc
        return carry
    lax.fori_loop(0, _RPT, unum_body, 0)
    pltpu.sync_copy(num_v, x_out.at[bs, pl.ds(26 * _D, _D)])

    def inum_body(r, carry):
        ivec = in_v[pl.ds(pl.multiple_of(r * _D, _D), _D)]
        acc = bin_v[...]
        for k in range(_INUM):
            acc = acc + ivec[k] * win_v[pl.ds(k * _D, _D)]
        num_v[r, :] = acc
        return carry
    lax.fori_loop(0, _RPT, inum_body, 0)
    pltpu.sync_copy(num_v, y_out.at[bs, pl.ds(_NI * _D, _D)])

    feats = []
    for f in range(_NU):
        feats.append((ucat, f * _B, f * _VU, utf,
                      x_out.at[bs, pl.ds(f * _D, _D)]))
    for f in range(_NC):
        feats.append((ccat, f * _B, f * _VC, ctf,
                      x_out.at[bs, pl.ds((_NU + f) * _D, _D)]))
    feats.append((icat, 0, 0, it0, y_out.at[bs, pl.ds(0, _D)]))
    for f in range(1, _NI):
        feats.append((icat, f * _B, (f - 1) * _VIR, irf,
                      y_out.at[bs, pl.ds(f * _D, _D)]))

    def prep_issue(i, idx_v, emb_v, sem):
        srcarr, srcoff, idxoff, table, _ = feats[i]
        pltpu.sync_copy(srcarr.at[pl.ds(srcoff + b0, _RPT)], idx_v)
        if idxoff:
            for c in range(_RPT // 16):
                sl = pl.ds(c * 16, 16)
                idx_v[sl] = idx_v[sl] + idxoff
        pltpu.async_copy(table.at[idx_v], emb_v, sem)

    nf = len(feats)
    prep_issue(0, idxA, embA, semA)
    for f in range(1, nf + 1):
        if f < nf:
            if f % 2 == 1:
                prep_issue(f, idxB, embB, semB)
            else:
                prep_issue(f, idxA, embA, semA)
        if (f - 1) % 2 == 0:
            pltpu.make_async_copy(it0.at[pl.ds(0, _RPT)], embA, semA).wait()
            pltpu.sync_copy(embA, feats[f - 1][4])
        else:
            pltpu.make_async_copy(it0.at[pl.ds(0, _RPT)], embB, semB).wait()
            pltpu.sync_copy(embB, feats[f - 1][4])

    pltpu.sync_copy(histf.at[pl.ds(b0 * _HL, _RPT * _HL)], ids_all)
    pltpu.sync_copy(hmaskf.at[pl.ds(b0 * _HL, _RPT * _HL)], hmask_all)

    def issue_stage(s, buf, sem):
        for k in range(_RPS):
            o = pl.multiple_of(s * _SID + k * _HL, 8)
            pltpu.async_copy(it0.at[ids_all.at[pl.ds(o, 128)]],
                             buf.at[pl.ds(k * _HL, 128)], sem)
            pltpu.async_copy(it0.at[ids_all.at[pl.ds(o + 128, _HL - 128)]],
                             buf.at[pl.ds(k * _HL + 128, _HL - 128)], sem)

    def drain_stage(buf, sem):
        for k in range(_RPS):
            pltpu.make_async_copy(it0.at[pl.ds(0, 128)],
                                  buf.at[pl.ds(k * _HL, 128)], sem).wait()
            pltpu.make_async_copy(it0.at[pl.ds(0, _HL - 128)],
                                  buf.at[pl.ds(k * _HL + 128, _HL - 128)],
                                  sem).wait()

    def compute_stage(s, buf):
        for k in range(_RPS):
            mbase = s * _SID + k * _HL
            zv = jnp.zeros((16,), jnp.float32)

            def acc_body(c, carry2):
                a, ms = carry2
                mvec = hmask_all[pl.ds(pl.multiple_of(mbase + c * 16, 8), 16)]
                base = k * _HL + c * 16
                for j in range(16):
                    mj = mvec[j]
                    a = a + buf[base + j, :] * mj
                    ms = ms + mj
                return (a, ms)
            a, ms = lax.fori_loop(0, 12, acc_body, (zv, jnp.float32(0.0)))
            mvec = hmask_all[pl.ds(pl.multiple_of(mbase + 192, 8), 16)]
            for j in range(8):
                mj = mvec[j]
                a = a + buf[k * _HL + 192 + j, :] * mj
                ms = ms + mj
            pool_v[s * _RPS + k, :] = a / jnp.maximum(ms, 1e-6)

    issue_stage(0, hrA, semA)

    def hist_loop(t, carry):
        sA = 2 * t
        sB = 2 * t + 1
        issue_stage(sB, hrB, semB)
        drain_stage(hrA, semA)
        compute_stage(sA, hrA)
        issue_stage(lax.rem(sA + 2, _NST), hrA, semA)
        drain_stage(hrB, semB)
        compute_stage(sB, hrB)
        return carry
    lax.fori_loop(0, _NST // 2, hist_loop, 0)
    drain_stage(hrA, semA)

    pltpu.sync_copy(pool_v, x_out.at[bs, pl.ds(27 * _D, _D)])


def _tc_body(x_ref, y_ref, wu1, bu1, wu2, bu2, wi1, bi1, wi2, bi2,
             u_ref, i_ref):
    f32 = jnp.float32
    xb = x_ref[...]
    h = jnp.maximum(
        jnp.dot(xb, wu1[...], preferred_element_type=f32) + bu1[...], 0.0)
    uu = jnp.dot(h, wu2[...], preferred_element_type=f32) + bu2[...]
    n = jnp.sqrt(jnp.sum(uu * uu, axis=-1, keepdims=True))
    u_ref[...] = uu / jnp.maximum(n, 1e-12)

    yb = y_ref[...]
    h2 = jnp.maximum(
        jnp.dot(yb, wi1[...], preferred_element_type=f32) + bi1[...], 0.0)
    ii = jnp.dot(h2, wi2[...], preferred_element_type=f32) + bi2[...]
    n2 = jnp.sqrt(jnp.sum(ii * ii, axis=-1, keepdims=True))
    i_ref[...] = ii / jnp.maximum(n2, 1e-12)


def kernel(user_cat, user_num, ctx_cat, hist_ids, hist_mask, item_cat,
           item_num, user_tables, ctx_tables, item_table0, item_tables_rest,
           Wun, bun, Win, bin, Wu1, bu1, Wu2, bu2, Wi1, bi1, Wi2, bi2):
    f32 = jnp.float32
    ucat_f = user_cat.T.astype(jnp.int32).reshape(-1)
    ccat_f = ctx_cat.T.astype(jnp.int32).reshape(-1)
    icat_f = item_cat.T.astype(jnp.int32).reshape(-1)
    hist_flat = hist_ids.reshape(-1).astype(jnp.int32)
    hmask_flat = hist_mask.reshape(-1)
    unum_pad = jnp.pad(user_num, ((0, 0), (0, _D - _UNUM))).reshape(-1)
    inum_pad = jnp.pad(item_num, ((0, 0), (0, _D - _INUM))).reshape(-1)
    ut_flat = user_tables.reshape(_NU * _VU, _D)
    ct_flat = ctx_tables.reshape(_NC * _VC, _D)
    ir_flat = item_tables_rest.reshape((_NI - 1) * _VIR, _D)

    mesh = plsc.VectorSubcoreMesh(core_axis_name="c", subcore_axis_name="s")
    sc = functools.partial(
        pl.kernel,
        mesh=mesh,
        compiler_params=pltpu.CompilerParams(use_tc_tiling_on_sc=False),
        out_type=[jax.ShapeDtypeStruct((_B, _UIN), f32),
                  jax.ShapeDtypeStruct((_B, _IIN), f32)],
        scratch_types=[
            pltpu.VMEM((_RPT,), jnp.int32),
            pltpu.VMEM((_RPT,), jnp.int32),
            pltpu.VMEM((_RPT, _D), f32),
            pltpu.VMEM((_RPT, _D), f32),
            pltpu.VMEM((_RPT * _HL,), jnp.int32),
            pltpu.VMEM((_RPT * _HL,), f32),
            pltpu.VMEM((_SID, _D), f32),
            pltpu.VMEM((_SID, _D), f32),
            pltpu.VMEM((_RPT, _D), f32),
            pltpu.VMEM((_RPT, _D), f32),
            pltpu.VMEM((_RPT * _D,), f32),
            pltpu.VMEM((_RPT * _D,), f32),
            pltpu.VMEM((_UNUM * _D,), f32),
            pltpu.VMEM((_D,), f32),
            pltpu.VMEM((_INUM * _D,), f32),
            pltpu.VMEM((_D,), f32),
            pltpu.SemaphoreType.DMA,
            pltpu.SemaphoreType.DMA,
        ],
    )(_sc_body)
    x, y = sc(ucat_f, ccat_f, icat_f, hist_flat, hmask_flat, unum_pad,
              inum_pad, Wun.reshape(-1), bun, Win.reshape(-1), bin,
              ut_flat, ct_flat, item_table0, ir_flat)

    bm = 1024
    grid = _B // bm
    full = lambda i: (0, 0)
    u, i = pl.pallas_call(
        _tc_body,
        grid=(grid,),
        in_specs=[
            pl.BlockSpec((bm, _UIN), lambda i: (i, 0)),
            pl.BlockSpec((bm, _IIN), lambda i: (i, 0)),
            pl.BlockSpec((_UIN, _HID), full),
            pl.BlockSpec((1, _HID), full),
            pl.BlockSpec((_HID, _TOW), full),
            pl.BlockSpec((1, _TOW), full),
            pl.BlockSpec((_IIN, _HID), full),
            pl.BlockSpec((1, _HID), full),
            pl.BlockSpec((_HID, _TOW), full),
            pl.BlockSpec((1, _TOW), full),
        ],
        out_specs=[pl.BlockSpec((bm, _TOW), lambda i: (i, 0)),
                   pl.BlockSpec((bm, _TOW), lambda i: (i, 0))],
        out_shape=[jax.ShapeDtypeStruct((_B, _TOW), f32),
                   jax.ShapeDtypeStruct((_B, _TOW), f32)],
    )(x, y, Wu1, bu1.reshape(1, _HID), Wu2, bu2.reshape(1, _TOW),
      Wi1, bi1.reshape(1, _HID), Wi2, bi2.reshape(1, _TOW))
    return (u, i)
